# baseline (speedup 1.0000x reference)
import functools

import jax
import jax.numpy as jnp
from jax import lax
from jax.experimental import pallas as pl
from jax.experimental.pallas import tpu as pltpu
from jax.experimental.pallas import tpu_sc as plsc

NC = 2
NS = 16
LANES = 16
CHUNK = 128


def _mm_body(x_ref, w_ref, o_ref):
    o_ref[...] = jnp.dot(x_ref[...], w_ref[...], preferred_element_type=jnp.float32)


def _combine_body(p_ref, b_ref, o_ref):
    o_ref[...] = p_ref[0] + p_ref[1] + b_ref[...]


def _make_sc_agg(n_nodes, chunks_per_tile, d):
    acc_rows = n_nodes
    rows_per_tile = (n_nodes // NS) & ~7
    rows_tail = n_nodes - NS * rows_per_tile
    mesh = plsc.VectorSubcoreMesh(core_axis_name="c", subcore_axis_name="s")

    @functools.partial(
        pl.kernel,
        out_type=jax.ShapeDtypeStruct((NC, n_nodes, d), jnp.float32),
        mesh=mesh,
        scratch_types=[
            pltpu.VMEM((CHUNK,), jnp.int32),
            pltpu.VMEM((CHUNK,), jnp.int32),
            pltpu.VMEM((CHUNK, d), jnp.float32),
            pltpu.VMEM_SHARED((acc_rows, d), jnp.float32),
            pltpu.SemaphoreType.DMA,
        ],
    )
    def sc_agg(y_hbm, src_hbm, dst_hbm, out_hbm,
               src_v, dst_v, buf_v, acc_sh, sem):
        c = lax.axis_index("c")
        s = lax.axis_index("s")
        w = c * NS + s

        def zero_row(i, carry):
            for j in range(d // LANES):
                buf_v[i, pl.ds(j * LANES, LANES)] = jnp.zeros((LANES,), jnp.float32)
            return carry
        lax.fori_loop(0, CHUNK, zero_row, 0)

        row_base = s * rows_per_tile
        n_full = rows_per_tile // CHUNK
        for k in range(n_full):
            pltpu.sync_copy(buf_v, acc_sh.at[pl.ds(row_base + k * CHUNK, CHUNK)])
        tail = rows_per_tile - n_full * CHUNK
        if tail:
            pltpu.sync_copy(buf_v.at[pl.ds(0, tail)],
                            acc_sh.at[pl.ds(row_base + n_full * CHUNK, tail)])
        if rows_tail:
            @pl.when(s == NS - 1)
            def _zero_last_rows():
                pltpu.sync_copy(buf_v.at[pl.ds(0, rows_tail)],
                                acc_sh.at[pl.ds(NS * rows_per_tile, rows_tail)])
        plsc.subcore_barrier()

        base0 = w * chunks_per_tile * CHUNK

        def chunk_one(i, carry):
            eb = base0 + i * CHUNK
            pltpu.sync_copy(src_hbm.at[pl.ds(eb, CHUNK)], src_v)
            pltpu.sync_copy(dst_hbm.at[pl.ds(eb, CHUNK)], dst_v)
            pltpu.async_copy(y_hbm.at[src_v], buf_v, sem).wait()
            pltpu.sync_copy(buf_v, acc_sh.at[dst_v], add=True)
            return carry
        lax.fori_loop(0, chunks_per_tile, chunk_one, 0)

        plsc.subcore_barrier()
        pltpu.sync_copy(acc_sh.at[pl.ds(row_base, rows_per_tile)],
                        out_hbm.at[c, pl.ds(row_base, rows_per_tile)])
        if rows_tail:
            @pl.when(s == NS - 1)
            def _copy_last_rows():
                pltpu.sync_copy(acc_sh.at[pl.ds(NS * rows_per_tile, rows_tail)],
                                out_hbm.at[c, pl.ds(NS * rows_per_tile, rows_tail)])

    return sc_agg


def kernel(x, edge_index, W, b):
    n_nodes, d_in = x.shape
    d_out = W.shape[1]
    n_edges = edge_index.shape[1]

    src = edge_index[1].astype(jnp.int32)
    dst = edge_index[0].astype(jnp.int32)

    nw = NC * NS
    cpt = -(-n_edges // (nw * CHUNK))
    n_pad = nw * cpt * CHUNK - n_edges
    src_p = jnp.concatenate([src, jnp.full((n_pad,), n_nodes, jnp.int32)])
    dst_p = jnp.concatenate([dst, jnp.arange(n_pad, dtype=jnp.int32) % n_nodes])

    row_blk = 1024
    mm_rows = -(-(n_nodes + 1) // row_blk) * row_blk
    x_p = jnp.concatenate(
        [x, jnp.zeros((mm_rows - n_nodes, d_in), jnp.float32)])
    y = pl.pallas_call(
        _mm_body,
        grid=(mm_rows // row_blk,),
        in_specs=[pl.BlockSpec((row_blk, d_in), lambda i: (i, 0)),
                  pl.BlockSpec((d_in, d_out), lambda i: (0, 0))],
        out_specs=pl.BlockSpec((row_blk, d_out), lambda i: (i, 0)),
        out_shape=jax.ShapeDtypeStruct((mm_rows, d_out), jnp.float32),
    )(x_p, W)

    partials = _make_sc_agg(n_nodes, cpt, d_out)(y, src_p, dst_p)

    cb_blk = 1000
    out = pl.pallas_call(
        _combine_body,
        grid=(n_nodes // cb_blk,),
        in_specs=[pl.BlockSpec((NC, cb_blk, d_out), lambda i: (0, i, 0)),
                  pl.BlockSpec((1, d_out), lambda i: (0, 0))],
        out_specs=pl.BlockSpec((cb_blk, d_out), lambda i: (i, 0)),
        out_shape=jax.ShapeDtypeStruct((n_nodes, d_out), jnp.float32),
    )(partials, b.reshape(1, d_out))
    return out


# speedup vs baseline: 2.4019x; 2.4019x over previous
import functools

import jax
import jax.numpy as jnp
from jax import lax
from jax.experimental import pallas as pl
from jax.experimental.pallas import tpu as pltpu
from jax.experimental.pallas import tpu_sc as plsc

NC = 2
NS = 16
LANES = 16
CHUNK = 128
DEPTH = 3


def _mm_body(x_ref, w_ref, o_ref):
    o_ref[...] = jnp.dot(x_ref[...], w_ref[...], preferred_element_type=jnp.float32)


def _combine_body(p_ref, b_ref, o_ref):
    o_ref[...] = p_ref[0] + p_ref[1] + b_ref[...]


def _make_sc_agg(n_nodes, n_edges, d):
    nw = NC * NS
    edges_per_tile = n_edges // nw
    full_chunks = edges_per_tile // CHUNK
    rem = edges_per_tile - full_chunks * CHUNK
    groups = full_chunks // DEPTH
    group_rem = full_chunks - groups * DEPTH
    rows_per_tile = (n_nodes // NS) & ~7
    rows_tail = n_nodes - NS * rows_per_tile
    mesh = plsc.VectorSubcoreMesh(core_axis_name="c", subcore_axis_name="s")

    @functools.partial(
        pl.kernel,
        out_type=jax.ShapeDtypeStruct((NC, n_nodes, d), jnp.float32),
        mesh=mesh,
        scratch_types=(
            [pltpu.VMEM((CHUNK,), jnp.int32) for _ in range(2 * DEPTH)]
            + [pltpu.VMEM((CHUNK, d), jnp.float32) for _ in range(DEPTH)]
            + [pltpu.VMEM((rem,), jnp.int32), pltpu.VMEM((rem,), jnp.int32)]
            + [pltpu.VMEM_SHARED((n_nodes, d), jnp.float32)]
            + [pltpu.SemaphoreType.DMA for _ in range(3 * DEPTH)]
        ),
    )
    def sc_agg(y_hbm, src_hbm, dst_hbm, out_hbm, *scr):
        srcs = scr[0:2 * DEPTH:2]
        dsts = scr[1:2 * DEPTH:2]
        bufs = scr[2 * DEPTH:3 * DEPTH]
        src_r, dst_r = scr[3 * DEPTH:3 * DEPTH + 2]
        acc_sh = scr[3 * DEPTH + 2]
        sems = scr[3 * DEPTH + 3:]
        sem_i = sems[0:DEPTH]
        sem_g = sems[DEPTH:2 * DEPTH]
        sem_s = sems[2 * DEPTH:3 * DEPTH]
        c = lax.axis_index("c")
        s = lax.axis_index("s")

        buf_z = bufs[0]

        def zero_row(i, carry):
            for j in range(d // LANES):
                buf_z[i, pl.ds(j * LANES, LANES)] = jnp.zeros((LANES,), jnp.float32)
            return carry
        lax.fori_loop(0, CHUNK, zero_row, 0)

        row_base = s * rows_per_tile
        n_full = rows_per_tile // CHUNK
        for k in range(n_full):
            pltpu.sync_copy(buf_z, acc_sh.at[pl.ds(row_base + k * CHUNK, CHUNK)])
        tail = rows_per_tile - n_full * CHUNK
        if tail:
            pltpu.sync_copy(buf_z.at[pl.ds(0, tail)],
                            acc_sh.at[pl.ds(row_base + n_full * CHUNK, tail)])
        if rows_tail:
            @pl.when(s == NS - 1)
            def _zero_last_rows():
                pltpu.sync_copy(buf_z.at[pl.ds(0, rows_tail)],
                                acc_sh.at[pl.ds(NS * rows_per_tile, rows_tail)])
        plsc.subcore_barrier()

        base0 = (c * NS + s) * edges_per_tile

        def chunk_group(i, carry):
            ih_s, ih_d, gh = [], [], []
            for k in range(DEPTH):
                eb = base0 + (DEPTH * i + k) * CHUNK
                ih_s.append(pltpu.async_copy(
                    src_hbm.at[pl.ds(eb, CHUNK)], srcs[k], sem_i[k]))
                ih_d.append(pltpu.async_copy(
                    dst_hbm.at[pl.ds(eb, CHUNK)], dsts[k], sem_i[k]))
            for k in range(DEPTH):
                ih_s[k].wait()
                gh.append(pltpu.async_copy(y_hbm.at[srcs[k]], bufs[k], sem_g[k]))
            sh = []
            for k in range(DEPTH):
                gh[k].wait()
                ih_d[k].wait()
                sh.append(pltpu.async_copy(
                    bufs[k], acc_sh.at[dsts[k]], sem_s[k], add=True))
            for k in range(DEPTH):
                sh[k].wait()
            return carry
        lax.fori_loop(0, groups, chunk_group, 0)

        for k in range(group_rem):
            eb = base0 + (groups * DEPTH + k) * CHUNK
            pltpu.sync_copy(src_hbm.at[pl.ds(eb, CHUNK)], srcs[0])
            pltpu.sync_copy(dst_hbm.at[pl.ds(eb, CHUNK)], dsts[0])
            pltpu.async_copy(y_hbm.at[srcs[0]], bufs[0], sem_g[0]).wait()
            pltpu.sync_copy(bufs[0], acc_sh.at[dsts[0]], add=True)

        if rem:
            eb = base0 + full_chunks * CHUNK
            pltpu.sync_copy(src_hbm.at[pl.ds(eb, rem)], src_r)
            pltpu.sync_copy(dst_hbm.at[pl.ds(eb, rem)], dst_r)
            pltpu.async_copy(y_hbm.at[src_r], bufs[0].at[pl.ds(0, rem)],
                             sem_g[0]).wait()
            pltpu.sync_copy(bufs[0].at[pl.ds(0, rem)], acc_sh.at[dst_r], add=True)

        plsc.subcore_barrier()
        pltpu.sync_copy(acc_sh.at[pl.ds(row_base, rows_per_tile)],
                        out_hbm.at[c, pl.ds(row_base, rows_per_tile)])
        if rows_tail:
            @pl.when(s == NS - 1)
            def _copy_last_rows():
                pltpu.sync_copy(acc_sh.at[pl.ds(NS * rows_per_tile, rows_tail)],
                                out_hbm.at[c, pl.ds(NS * rows_per_tile, rows_tail)])

    return sc_agg


def kernel(x, edge_index, W, b):
    n_nodes, d_in = x.shape
    d_out = W.shape[1]
    n_edges = edge_index.shape[1]

    src = edge_index[1].astype(jnp.int32)
    dst = edge_index[0].astype(jnp.int32)

    row_blk = 1000
    y = pl.pallas_call(
        _mm_body,
        grid=(n_nodes // row_blk,),
        in_specs=[pl.BlockSpec((row_blk, d_in), lambda i: (i, 0)),
                  pl.BlockSpec((d_in, d_out), lambda i: (0, 0))],
        out_specs=pl.BlockSpec((row_blk, d_out), lambda i: (i, 0)),
        out_shape=jax.ShapeDtypeStruct((n_nodes, d_out), jnp.float32),
    )(x, W)

    partials = _make_sc_agg(n_nodes, n_edges, d_out)(y, src, dst)

    out = pl.pallas_call(
        _combine_body,
        grid=(n_nodes // row_blk,),
        in_specs=[pl.BlockSpec((NC, row_blk, d_out), lambda i: (0, i, 0)),
                  pl.BlockSpec((1, d_out), lambda i: (0, 0))],
        out_specs=pl.BlockSpec((row_blk, d_out), lambda i: (i, 0)),
        out_shape=jax.ShapeDtypeStruct((n_nodes, d_out), jnp.float32),
    )(partials, b.reshape(1, d_out))
    return out


# speedup vs baseline: 2.6027x; 1.0836x over previous
import functools

import jax
import jax.numpy as jnp
from jax import lax
from jax.experimental import pallas as pl
from jax.experimental.pallas import tpu as pltpu
from jax.experimental.pallas import tpu_sc as plsc

NC = 2
NS = 16
LANES = 16
CHUNK = 128
DEPTH = 3


def _mm_body(x_ref, w_ref, o_ref):
    o_ref[...] = jnp.dot(x_ref[...], w_ref[...], preferred_element_type=jnp.float32)


def _combine_body(p_ref, b_ref, o_ref):
    o_ref[...] = p_ref[0] + p_ref[1] + b_ref[...]


def _make_sc_agg(n_nodes, n_edges, d):
    nw = NC * NS
    edges_per_tile = n_edges // nw
    full_chunks = edges_per_tile // CHUNK
    rem = edges_per_tile - full_chunks * CHUNK
    groups = full_chunks // (2 * DEPTH)
    group_rem = full_chunks - groups * 2 * DEPTH
    rows_per_tile = (n_nodes // NS) & ~7
    rows_tail = n_nodes - NS * rows_per_tile
    mesh = plsc.VectorSubcoreMesh(core_axis_name="c", subcore_axis_name="s")

    @functools.partial(
        pl.kernel,
        out_type=jax.ShapeDtypeStruct((NC, n_nodes, d), jnp.float32),
        mesh=mesh,
        scratch_types=(
            [pltpu.VMEM((CHUNK,), jnp.int32) for _ in range(4 * DEPTH)]
            + [pltpu.VMEM((CHUNK, d), jnp.float32) for _ in range(DEPTH)]
            + [pltpu.VMEM((rem,), jnp.int32), pltpu.VMEM((rem,), jnp.int32)]
            + [pltpu.VMEM_SHARED((n_nodes, d), jnp.float32)]
            + [pltpu.SemaphoreType.DMA for _ in range(6 * DEPTH)]
        ),
    )
    def sc_agg(y_hbm, src_hbm, dst_hbm, out_hbm, *scr):
        srcs = scr[0:4 * DEPTH:2]
        dsts = scr[1:4 * DEPTH:2]
        bufs = scr[4 * DEPTH:5 * DEPTH]
        src_r, dst_r = scr[5 * DEPTH:5 * DEPTH + 2]
        acc_sh = scr[5 * DEPTH + 2]
        sems = scr[5 * DEPTH + 3:]
        sem_is = sems[0:2 * DEPTH]
        sem_id = sems[2 * DEPTH:4 * DEPTH]
        sem_g = sems[4 * DEPTH:5 * DEPTH]
        sem_s = sems[5 * DEPTH:6 * DEPTH]
        c = lax.axis_index("c")
        s = lax.axis_index("s")

        buf_z = bufs[0]

        def zero_row(i, carry):
            for j in range(d // LANES):
                buf_z[i, pl.ds(j * LANES, LANES)] = jnp.zeros((LANES,), jnp.float32)
            return carry
        lax.fori_loop(0, CHUNK, zero_row, 0)

        row_base = s * rows_per_tile
        n_full = rows_per_tile // CHUNK
        for k in range(n_full):
            pltpu.sync_copy(buf_z, acc_sh.at[pl.ds(row_base + k * CHUNK, CHUNK)])
        tail = rows_per_tile - n_full * CHUNK
        if tail:
            pltpu.sync_copy(buf_z.at[pl.ds(0, tail)],
                            acc_sh.at[pl.ds(row_base + n_full * CHUNK, tail)])
        if rows_tail:
            @pl.when(s == NS - 1)
            def _zero_last_rows():
                pltpu.sync_copy(buf_z.at[pl.ds(0, rows_tail)],
                                acc_sh.at[pl.ds(NS * rows_per_tile, rows_tail)])
        plsc.subcore_barrier()

        base0 = (c * NS + s) * edges_per_tile

        def chunk_group(i, carry):
            ih_s, ih_d = [], []
            for j in range(2 * DEPTH):
                eb = base0 + (2 * DEPTH * i + j) * CHUNK
                ih_s.append(pltpu.async_copy(
                    src_hbm.at[pl.ds(eb, CHUNK)], srcs[j], sem_is[j]))
                ih_d.append(pltpu.async_copy(
                    dst_hbm.at[pl.ds(eb, CHUNK)], dsts[j], sem_id[j]))
            gh = []
            for k in range(DEPTH):
                ih_s[k].wait()
                gh.append(pltpu.async_copy(y_hbm.at[srcs[k]], bufs[k], sem_g[k]))
            sh = []
            for k in range(DEPTH):
                gh[k].wait()
                ih_d[k].wait()
                sh.append(pltpu.async_copy(
                    bufs[k], acc_sh.at[dsts[k]], sem_s[k], add=True))
            gh2 = []
            for k in range(DEPTH):
                sh[k].wait()
                ih_s[DEPTH + k].wait()
                gh2.append(pltpu.async_copy(
                    y_hbm.at[srcs[DEPTH + k]], bufs[k], sem_g[k]))
            sh2 = []
            for k in range(DEPTH):
                gh2[k].wait()
                ih_d[DEPTH + k].wait()
                sh2.append(pltpu.async_copy(
                    bufs[k], acc_sh.at[dsts[DEPTH + k]], sem_s[k], add=True))
            for k in range(DEPTH):
                sh2[k].wait()
            return carry
        lax.fori_loop(0, groups, chunk_group, 0)

        for k in range(group_rem):
            eb = base0 + (groups * 2 * DEPTH + k) * CHUNK
            pltpu.sync_copy(src_hbm.at[pl.ds(eb, CHUNK)], srcs[0])
            pltpu.sync_copy(dst_hbm.at[pl.ds(eb, CHUNK)], dsts[0])
            pltpu.async_copy(y_hbm.at[srcs[0]], bufs[0], sem_g[0]).wait()
            pltpu.sync_copy(bufs[0], acc_sh.at[dsts[0]], add=True)

        if rem:
            eb = base0 + full_chunks * CHUNK
            pltpu.sync_copy(src_hbm.at[pl.ds(eb, rem)], src_r)
            pltpu.sync_copy(dst_hbm.at[pl.ds(eb, rem)], dst_r)
            pltpu.async_copy(y_hbm.at[src_r], bufs[0].at[pl.ds(0, rem)],
                             sem_g[0]).wait()
            pltpu.sync_copy(bufs[0].at[pl.ds(0, rem)], acc_sh.at[dst_r], add=True)

        plsc.subcore_barrier()
        pltpu.sync_copy(acc_sh.at[pl.ds(row_base, rows_per_tile)],
                        out_hbm.at[c, pl.ds(row_base, rows_per_tile)])
        if rows_tail:
            @pl.when(s == NS - 1)
            def _copy_last_rows():
                pltpu.sync_copy(acc_sh.at[pl.ds(NS * rows_per_tile, rows_tail)],
                                out_hbm.at[c, pl.ds(NS * rows_per_tile, rows_tail)])

    return sc_agg


def kernel(x, edge_index, W, b):
    n_nodes, d_in = x.shape
    d_out = W.shape[1]
    n_edges = edge_index.shape[1]

    src = edge_index[1].astype(jnp.int32)
    dst = edge_index[0].astype(jnp.int32)

    row_blk = 1000
    y = pl.pallas_call(
        _mm_body,
        grid=(n_nodes // row_blk,),
        in_specs=[pl.BlockSpec((row_blk, d_in), lambda i: (i, 0)),
                  pl.BlockSpec((d_in, d_out), lambda i: (0, 0))],
        out_specs=pl.BlockSpec((row_blk, d_out), lambda i: (i, 0)),
        out_shape=jax.ShapeDtypeStruct((n_nodes, d_out), jnp.float32),
    )(x, W)

    partials = _make_sc_agg(n_nodes, n_edges, d_out)(y, src, dst)

    out = pl.pallas_call(
        _combine_body,
        grid=(n_nodes // row_blk,),
        in_specs=[pl.BlockSpec((NC, row_blk, d_out), lambda i: (0, i, 0)),
                  pl.BlockSpec((1, d_out), lambda i: (0, 0))],
        out_specs=pl.BlockSpec((row_blk, d_out), lambda i: (i, 0)),
        out_shape=jax.ShapeDtypeStruct((n_nodes, d_out), jnp.float32),
    )(partials, b.reshape(1, d_out))
    return out


# speedup vs baseline: 3.3751x; 1.2968x over previous
import functools

import jax
import jax.numpy as jnp
from jax import lax
from jax.experimental import pallas as pl
from jax.experimental.pallas import tpu as pltpu
from jax.experimental.pallas import tpu_sc as plsc

NC = 2
NS = 16
LANES = 16
CHUNK = 128
DEPTH = 3


def _mm_body(x_ref, w_ref, o_ref):
    o_ref[...] = jnp.dot(x_ref[...], w_ref[...], preferred_element_type=jnp.float32)


def _combine_body(p_ref, b_ref, o_ref):
    o_ref[...] = p_ref[0] + p_ref[1] + b_ref[...]


def _make_sc_agg(n_nodes, n_edges, d):
    nw = NC * NS
    edges_per_tile = n_edges // nw
    full_chunks = edges_per_tile // CHUNK
    rem = edges_per_tile - full_chunks * CHUNK
    groups = full_chunks // (2 * DEPTH)
    group_rem = full_chunks - groups * 2 * DEPTH
    rows_per_tile = (n_nodes // NS) & ~7
    rows_tail = n_nodes - NS * rows_per_tile
    mesh = plsc.VectorSubcoreMesh(core_axis_name="c", subcore_axis_name="s")

    @functools.partial(
        pl.kernel,
        out_type=jax.ShapeDtypeStruct((NC, n_nodes, d), jnp.float32),
        mesh=mesh,
        scratch_types=(
            [pltpu.VMEM((CHUNK,), jnp.int32) for _ in range(4 * DEPTH)]
            + [pltpu.VMEM((CHUNK, d), jnp.float32) for _ in range(DEPTH)]
            + [pltpu.VMEM((rem,), jnp.int32), pltpu.VMEM((rem,), jnp.int32)]
            + [pltpu.VMEM_SHARED((n_nodes, d), jnp.float32)]
            + [pltpu.SemaphoreType.DMA for _ in range(6 * DEPTH)]
        ),
    )
    def sc_agg(y_hbm, src_hbm, dst_hbm, out_hbm, *scr):
        srcs = scr[0:4 * DEPTH:2]
        dsts = scr[1:4 * DEPTH:2]
        bufs = scr[4 * DEPTH:5 * DEPTH]
        src_r, dst_r = scr[5 * DEPTH:5 * DEPTH + 2]
        acc_sh = scr[5 * DEPTH + 2]
        sems = scr[5 * DEPTH + 3:]
        sem_is = sems[0:2 * DEPTH]
        sem_id = sems[2 * DEPTH:4 * DEPTH]
        sem_g = sems[4 * DEPTH:5 * DEPTH]
        sem_s = sems[5 * DEPTH:6 * DEPTH]
        c = lax.axis_index("c")
        s = lax.axis_index("s")

        buf_z = bufs[0]

        def zero_row(i, carry):
            for j in range(d // LANES):
                buf_z[i, pl.ds(j * LANES, LANES)] = jnp.zeros((LANES,), jnp.float32)
            return carry
        lax.fori_loop(0, CHUNK, zero_row, 0)

        row_base = s * rows_per_tile
        n_full = rows_per_tile // CHUNK
        for k in range(n_full):
            pltpu.sync_copy(buf_z, acc_sh.at[pl.ds(row_base + k * CHUNK, CHUNK)])
        tail = rows_per_tile - n_full * CHUNK
        if tail:
            pltpu.sync_copy(buf_z.at[pl.ds(0, tail)],
                            acc_sh.at[pl.ds(row_base + n_full * CHUNK, tail)])
        if rows_tail:
            @pl.when(s == NS - 1)
            def _zero_last_rows():
                pltpu.sync_copy(buf_z.at[pl.ds(0, rows_tail)],
                                acc_sh.at[pl.ds(NS * rows_per_tile, rows_tail)])
        plsc.subcore_barrier()

        base0 = (c * NS + s) * edges_per_tile
        nslots = 2 * DEPTH

        ih_s = [None] * nslots
        ih_d = [None] * nslots
        gh = [None] * DEPTH
        sh = [None] * DEPTH

        def load_idx(j):
            m = j % nslots
            eb = base0 + j * CHUNK
            ih_s[m] = pltpu.async_copy(
                src_hbm.at[pl.ds(eb, CHUNK)], srcs[m], sem_is[m])
            ih_d[m] = pltpu.async_copy(
                dst_hbm.at[pl.ds(eb, CHUNK)], dsts[m], sem_id[m])

        def start_gather(j):
            k, m = j % DEPTH, j % nslots
            if j >= DEPTH:
                sh[k].wait()
                if j + DEPTH < full_chunks:
                    load_idx(j + DEPTH)
            ih_s[m].wait()
            gh[k] = pltpu.async_copy(y_hbm.at[srcs[m]], bufs[k], sem_g[k])

        def start_scatter(j):
            k, m = j % DEPTH, j % nslots
            gh[k].wait()
            ih_d[m].wait()
            sh[k] = pltpu.async_copy(
                bufs[k], acc_sh.at[dsts[m]], sem_s[k], add=True)

        for j in range(min(nslots, full_chunks)):
            load_idx(j)
        for j in range(min(DEPTH, full_chunks)):
            start_gather(j)
        for j in range(full_chunks):
            start_scatter(j)
            if j + DEPTH < full_chunks:
                start_gather(j + DEPTH)
        for j in range(max(0, full_chunks - DEPTH), full_chunks):
            sh[j % DEPTH].wait()

        if rem:
            eb = base0 + full_chunks * CHUNK
            pltpu.sync_copy(src_hbm.at[pl.ds(eb, rem)], src_r)
            pltpu.sync_copy(dst_hbm.at[pl.ds(eb, rem)], dst_r)
            pltpu.async_copy(y_hbm.at[src_r], bufs[0].at[pl.ds(0, rem)],
                             sem_g[0]).wait()
            pltpu.sync_copy(bufs[0].at[pl.ds(0, rem)], acc_sh.at[dst_r], add=True)

        plsc.subcore_barrier()
        pltpu.sync_copy(acc_sh.at[pl.ds(row_base, rows_per_tile)],
                        out_hbm.at[c, pl.ds(row_base, rows_per_tile)])
        if rows_tail:
            @pl.when(s == NS - 1)
            def _copy_last_rows():
                pltpu.sync_copy(acc_sh.at[pl.ds(NS * rows_per_tile, rows_tail)],
                                out_hbm.at[c, pl.ds(NS * rows_per_tile, rows_tail)])

    return sc_agg


def kernel(x, edge_index, W, b):
    n_nodes, d_in = x.shape
    d_out = W.shape[1]
    n_edges = edge_index.shape[1]

    src = edge_index[1].astype(jnp.int32)
    dst = edge_index[0].astype(jnp.int32)

    row_blk = 1000
    y = pl.pallas_call(
        _mm_body,
        grid=(n_nodes // row_blk,),
        in_specs=[pl.BlockSpec((row_blk, d_in), lambda i: (i, 0)),
                  pl.BlockSpec((d_in, d_out), lambda i: (0, 0))],
        out_specs=pl.BlockSpec((row_blk, d_out), lambda i: (i, 0)),
        out_shape=jax.ShapeDtypeStruct((n_nodes, d_out), jnp.float32),
    )(x, W)

    partials = _make_sc_agg(n_nodes, n_edges, d_out)(y, src, dst)

    out = pl.pallas_call(
        _combine_body,
        grid=(n_nodes // row_blk,),
        in_specs=[pl.BlockSpec((NC, row_blk, d_out), lambda i: (0, i, 0)),
                  pl.BlockSpec((1, d_out), lambda i: (0, 0))],
        out_specs=pl.BlockSpec((row_blk, d_out), lambda i: (i, 0)),
        out_shape=jax.ShapeDtypeStruct((n_nodes, d_out), jnp.float32),
    )(partials, b.reshape(1, d_out))
    return out
